# Initial kernel scaffold; baseline (speedup 1.0000x reference)
import jax
import jax.numpy as jnp
from jax.experimental import pallas as pl


def kernel(X, embed, W1, b1, gamma, beta, W2, b2):
    raise NotImplementedError("write your pallas kernel here")


# speedup vs baseline: 7.4931x; 7.4931x over previous
import functools

import jax
import jax.numpy as jnp
from jax import lax
from jax.experimental import pallas as pl
from jax.experimental.pallas import tpu as pltpu
from jax.experimental.pallas import tpu_sc as plsc

_D = 128
_B = 4096
_S = 200
_NC = 2
_NS = 16
_NW = _NC * _NS
_BPW = _B // _NW
_C1 = 104
_C2 = _S - _C1


def _make_pool():
    mesh = plsc.VectorSubcoreMesh(core_axis_name="c", subcore_axis_name="s")

    @functools.partial(
        pl.kernel,
        mesh=mesh,
        out_type=jax.ShapeDtypeStruct((_B, _D), jnp.float32),
        scratch_types=[
            pltpu.VMEM((_BPW * _S,), jnp.int32),
            pltpu.VMEM((_S, _D), jnp.float32),
            pltpu.VMEM((_BPW, _D), jnp.float32),
            pltpu.SemaphoreType.DMA,
        ],
    )
    def pool(xflat, embed, out, idx_v, rows_v, m_v, sem):
        wid = lax.axis_index("s") * _NC + lax.axis_index("c")
        base = wid * _BPW
        pltpu.sync_copy(xflat.at[pl.ds(base * _S, _BPW * _S)], idx_v)

        def per_b(b, carry):
            off = pl.multiple_of(b * _S, 8)
            cp1 = pltpu.async_copy(
                embed.at[idx_v.at[pl.ds(off, _C1)]], rows_v.at[pl.ds(0, _C1)], sem)
            cp2 = pltpu.async_copy(
                embed.at[idx_v.at[pl.ds(off + _C1, _C2)]], rows_v.at[pl.ds(_C1, _C2)], sem)
            cp1.wait()
            cp2.wait()

            def srow(s, accs):
                return tuple(accs[j] + rows_v[s, pl.ds(j * 16, 16)] for j in range(8))

            accs = lax.fori_loop(
                0, _S, srow, tuple(jnp.zeros((16,), jnp.float32) for _ in range(8)))
            for j in range(8):
                m_v[b, pl.ds(j * 16, 16)] = accs[j]
            return carry

        lax.fori_loop(0, _BPW, per_b, 0)
        pltpu.sync_copy(m_v, out.at[pl.ds(base, _BPW)])

    return pool


_pool = _make_pool()


def _mlp_body(m_ref, w1_ref, b1_ref, g_ref, be_ref, w2_ref, b2_ref, out_ref):
    m = m_ref[...] * (1.0 / _S)
    h = lax.dot_general(m, w1_ref[...], (((1,), (1,)), ((), ())),
                        preferred_element_type=jnp.float32) + b1_ref[...][None, :]
    mu = jnp.mean(h, axis=0)
    var = jnp.mean(jnp.square(h - mu), axis=0)
    hn = (h - mu) * lax.rsqrt(var + 1e-5) * g_ref[...][None, :] + be_ref[...][None, :]
    a = jnp.maximum(hn, 0.0)
    out_ref[...] = lax.dot_general(a, w2_ref[...], (((1,), (1,)), ((), ())),
                                   preferred_element_type=jnp.float32) + b2_ref[...][None, :]


@jax.jit
def kernel(X, embed, W1, b1, gamma, beta, W2, b2):
    msum = _pool(X.reshape(-1).astype(jnp.int32), embed)
    out = pl.pallas_call(
        _mlp_body,
        out_shape=jax.ShapeDtypeStruct((_B, W2.shape[0]), jnp.float32),
    )(msum, W1, b1, gamma, beta, W2, b2)
    return out


# speedup vs baseline: 12.9384x; 1.7267x over previous
import functools

import jax
import jax.numpy as jnp
from jax import lax
from jax.experimental import pallas as pl
from jax.experimental.pallas import tpu as pltpu
from jax.experimental.pallas import tpu_sc as plsc

_D = 128
_B = 4096
_S = 200
_NC = 2
_NS = 16
_NW = _NC * _NS
_BPW = _B // _NW
_C1 = 104
_C2 = _S - _C1


def _make_pool():
    mesh = plsc.VectorSubcoreMesh(core_axis_name="c", subcore_axis_name="s")

    @functools.partial(
        pl.kernel,
        mesh=mesh,
        out_type=jax.ShapeDtypeStruct((_B, _D), jnp.float32),
        scratch_types=[
            pltpu.VMEM((_BPW * _S,), jnp.int32),
            pltpu.VMEM((_S, _D), jnp.float32),
            pltpu.VMEM((_S, _D), jnp.float32),
            pltpu.VMEM((_BPW, _D), jnp.float32),
            pltpu.SemaphoreType.DMA,
            pltpu.SemaphoreType.DMA,
        ],
    )
    def pool(xflat, embed, out, idx_v, rows0, rows1, m_v, sem0, sem1):
        wid = lax.axis_index("s") * _NC + lax.axis_index("c")
        base = wid * _BPW
        pltpu.sync_copy(xflat.at[pl.ds(base * _S, _BPW * _S)], idx_v)

        def start(b, rows, sem):
            off = pl.multiple_of(b * _S, 8)
            pltpu.async_copy(
                embed.at[idx_v.at[pl.ds(off, _C1)]], rows.at[pl.ds(0, _C1)], sem)
            pltpu.async_copy(
                embed.at[idx_v.at[pl.ds(off + _C1, _C2)]], rows.at[pl.ds(_C1, _C2)], sem)

        def wait(rows, sem):
            pltpu.make_async_copy(embed.at[pl.ds(0, _S)], rows, sem).wait()

        def accumulate(rows, b):
            def srow(s4, accs):
                s0 = s4 * 4
                for dr in range(4):
                    accs = tuple(
                        accs[j] + rows[s0 + dr, pl.ds(j * 16, 16)] for j in range(8))
                return accs

            accs = lax.fori_loop(
                0, _S // 4, srow, tuple(jnp.zeros((16,), jnp.float32) for _ in range(8)))
            for j in range(8):
                m_v[b, pl.ds(j * 16, 16)] = accs[j]

        start(0, rows0, sem0)
        start(1, rows1, sem1)
        bufs = ((rows0, sem0), (rows1, sem1))

        def body(i, carry):
            for k in range(2):
                b = 2 * i + k
                rows, sem = bufs[k]
                wait(rows, sem)
                accumulate(rows, b)

                @pl.when(b + 2 < _BPW)
                def _():
                    start(b + 2, rows, sem)
            return carry

        lax.fori_loop(0, _BPW // 2, body, 0)
        pltpu.sync_copy(m_v, out.at[pl.ds(base, _BPW)])

    return pool


_pool = _make_pool()


def _mlp_body(m_ref, w1_ref, b1_ref, g_ref, be_ref, w2_ref, b2_ref, out_ref):
    m = m_ref[...] * (1.0 / _S)
    h = lax.dot_general(m, w1_ref[...], (((1,), (1,)), ((), ())),
                        preferred_element_type=jnp.float32) + b1_ref[...][None, :]
    mu = jnp.mean(h, axis=0)
    var = jnp.mean(jnp.square(h - mu), axis=0)
    hn = (h - mu) * lax.rsqrt(var + 1e-5) * g_ref[...][None, :] + be_ref[...][None, :]
    a = jnp.maximum(hn, 0.0)
    out_ref[...] = lax.dot_general(a, w2_ref[...], (((1,), (1,)), ((), ())),
                                   preferred_element_type=jnp.float32) + b2_ref[...][None, :]


@jax.jit
def kernel(X, embed, W1, b1, gamma, beta, W2, b2):
    msum = _pool(X.reshape(-1).astype(jnp.int32), embed)
    out = pl.pallas_call(
        _mlp_body,
        out_shape=jax.ShapeDtypeStruct((_B, W2.shape[0]), jnp.float32),
    )(msum, W1, b1, gamma, beta, W2, b2)
    return out
